# baseline (speedup 1.0000x reference)
import jax
import jax.numpy as jnp
from jax import lax
from jax.experimental import pallas as pl
from jax.experimental.pallas import tpu as pltpu
from jax.experimental.pallas import tpu_sc as plsc

N_NODES = 10000
N_EDGES = 160000
D = 256
NC, NS, L = 2, 16, 16
NW = NC * NS
NP = 10240
NE_PAD = 163840

_SC_PARAMS = pltpu.CompilerParams(needs_layout_passes=False)


def _sc_mesh():
    return plsc.VectorSubcoreMesh(core_axis_name="c", subcore_axis_name="s",
                                  num_cores=NC, num_subcores=NS)


def _bf16_bits(x):
    return lax.bitcast_convert_type(x.astype(jnp.bfloat16).astype(jnp.float32),
                                    jnp.int32)


def _pack_pair(even, odd):
    lo = lax.shift_right_logical(_bf16_bits(even), 16)
    hi = jnp.bitwise_and(_bf16_bits(odd), jnp.int32(-65536))
    return jnp.bitwise_or(lo, hi)


EROWS = N_EDGES // 128
EROWS_P = NE_PAD // 128
ERB = EROWS_P // 20


def _proj_body(x_ref, e_ref, wqe_ref, wqo_ref, wke_ref, wko_ref, wve_ref, wvo_ref,
               bqe_ref, bqo_ref, bke_ref, bko_ref, bve_ref, bvo_ref,
               qt_ref, kt_ref, vt_ref, s_ref, d_ref):
    x = x_ref[...]
    inv = 1.0 / (D ** 0.5)
    dn = (((1,), (1,)), ((), ()))

    def mm(w_ref, b_ref):
        return lax.dot_general(w_ref[...], x, dn,
                               preferred_element_type=jnp.float32) + b_ref[...]

    qt_ref[...] = _pack_pair(mm(wqe_ref, bqe_ref) * inv, mm(wqo_ref, bqo_ref) * inv)
    kt_ref[...] = _pack_pair(mm(wke_ref, bke_ref), mm(wko_ref, bko_ref))
    vt_ref[...] = _pack_pair(mm(wve_ref, bve_ref), mm(wvo_ref, bvo_ref))
    i = pl.program_id(0)
    e = e_ref[...]
    valid = i * ERB + lax.broadcasted_iota(jnp.int32, (ERB, 128), 0) < EROWS
    s_ref[...] = jnp.where(valid, e[0], 0)
    d_ref[...] = jnp.where(valid, e[1], 0)


def _project(emb, edge3, ws, bs):
    R = 512
    half = pl.BlockSpec((D // 2, D), lambda i: (0, 0))
    bcol = pl.BlockSpec((D // 2, 1), lambda i: (0, 0))
    outs = pl.BlockSpec((D // 2, R), lambda i: (0, i))
    epad = pl.BlockSpec((ERB, 128), lambda i: (i, 0))
    return pl.pallas_call(
        _proj_body,
        grid=(NP // R,),
        in_specs=[pl.BlockSpec((R, D), lambda i: (i, 0)),
                  pl.BlockSpec((2, ERB, 128), lambda i: (0, i, 0))]
                 + [half] * 6 + [bcol] * 6,
        out_specs=[outs] * 3 + [epad] * 2,
        out_shape=[jax.ShapeDtypeStruct((D // 2, NP), jnp.int32)] * 3
                  + [jax.ShapeDtypeStruct((EROWS_P, 128), jnp.int32)] * 2,
    )(emb, edge3, *ws, *bs)


C2 = 2048
NCH2 = NE_PAD // C2
GU2 = 8


def _scores_body(qt_hbm, kt_hbm, dst_hbm, src_hbm, sp_hbm,
                 qs0, qs1, qs2, qs3, ks0, ks1, ks2, ks3,
                 si0, di0, si1, di1, pb0, pb1,
                 semq, sem0, sem1, semp):
    cid = lax.axis_index("c")
    sid = lax.axis_index("s")
    wid = sid * NC + cid
    qslabs = (qs0, qs1, qs2, qs3)
    kslabs = (ks0, ks1, ks2, ks3)
    ibufs = ((si0, di0, sem0), (si1, di1, sem1))
    pbufs = (pb0, pb1)

    def issue(i, b):
        si, di, sem = ibufs[b]
        off = i * C2
        pltpu.async_copy(src_hbm.at[pl.ds(off, C2)], si, sem)
        pltpu.async_copy(dst_hbm.at[pl.ds(off, C2)], di, sem)

    def wait(b):
        si, di, sem = ibufs[b]
        pltpu.make_async_copy(src_hbm.at[pl.ds(0, C2)], si, sem).wait()
        pltpu.make_async_copy(dst_hbm.at[pl.ds(0, C2)], di, sem).wait()

    def compute(i, b):
        si, di, _ = ibufs[b]
        pbuf = pbufs[b]

        def grp(t, c2):
            for u in range(GU2):
                g = t * GU2 + u
                s16 = si[pl.ds(g * L, L)]
                d16 = di[pl.ds(g * L, L)]
                acc = jnp.zeros((L,), jnp.float32)
                for j in range(PAIRS):
                    qw = plsc.load_gather(qslabs[j], [d16])
                    kw = plsc.load_gather(kslabs[j], [s16])
                    qe = plsc.bitcast(lax.shift_left(qw, 16), jnp.float32)
                    ke = plsc.bitcast(lax.shift_left(kw, 16), jnp.float32)
                    qo = plsc.bitcast(jnp.bitwise_and(qw, jnp.int32(M_HI)), jnp.float32)
                    ko = plsc.bitcast(jnp.bitwise_and(kw, jnp.int32(M_HI)), jnp.float32)
                    acc = acc + qe * ke + qo * ko
                pbuf[pl.ds(g * L, L)] = acc
            return c2

        lax.fori_loop(0, C2 // (L * GU2), grp, 0, unroll=False)
        pltpu.async_copy(pbuf, sp_hbm.at[wid, pl.ds(i * C2, C2)], semp)

    def wait_pbuf(b):
        pltpu.make_async_copy(pbufs[b], sp_hbm.at[wid, pl.ds(0, C2)], semp).wait()

    cps = [pltpu.async_copy(qt_hbm.at[wid, pl.ds(j * NP, NP)], qslabs[j], semq)
           for j in range(PAIRS)]
    cps += [pltpu.async_copy(kt_hbm.at[wid, pl.ds(j * NP, NP)], kslabs[j], semq)
            for j in range(PAIRS)]
    issue(0, 0)
    for cp in cps:
        cp.wait()

    def outer(i2, c):
        a = 2 * i2
        issue(a + 1, 1)
        wait(0)

        @pl.when(i2 > 0)
        def _():
            wait_pbuf(0)

        compute(a, 0)

        @pl.when(a + 2 < NCH2)
        def _():
            issue(a + 2, 0)

        wait(1)

        @pl.when(i2 > 0)
        def _():
            wait_pbuf(1)

        compute(a + 1, 1)
        return c

    lax.fori_loop(0, NCH2 // 2, outer, 0, unroll=False)
    wait_pbuf(0)
    wait_pbuf(1)


def _scores(qt_slabs, kt_slabs, dst, src):
    return pl.kernel(
        _scores_body,
        out_type=jax.ShapeDtypeStruct((NW, NE_PAD), jnp.float32),
        mesh=_sc_mesh(),
        compiler_params=_SC_PARAMS,
        scratch_types=[pltpu.VMEM((NP,), jnp.int32)] * (2 * PAIRS) + [
            pltpu.VMEM((C2,), jnp.int32),
            pltpu.VMEM((C2,), jnp.int32),
            pltpu.VMEM((C2,), jnp.int32),
            pltpu.VMEM((C2,), jnp.int32),
            pltpu.VMEM((C2,), jnp.float32),
            pltpu.VMEM((C2,), jnp.float32),
            pltpu.SemaphoreType.DMA,
            pltpu.SemaphoreType.DMA,
            pltpu.SemaphoreType.DMA,
            pltpu.SemaphoreType.DMA,
        ],
    )(qt_slabs, kt_slabs, dst, src)


SM_ROWS = NE_PAD // 128


def _softmax_body(sp_ref, w_ref):
    s = jnp.sum(sp_ref[...], axis=0)
    rows = lax.broadcasted_iota(jnp.int32, (SM_ROWS, 128), 0)
    cols = lax.broadcasted_iota(jnp.int32, (SM_ROWS, 128), 1)
    valid = rows * 128 + cols < N_EDGES
    s = jnp.where(valid, s, -jnp.inf)
    m = jnp.max(s)
    e = jnp.where(valid, jnp.exp(s - m), 0.0)
    w_ref[...] = e / jnp.sum(e)


def _softmax(sparts):
    return pl.pallas_call(
        _softmax_body,
        out_shape=jax.ShapeDtypeStruct((SM_ROWS, 128), jnp.float32),
    )(sparts.reshape(NW, SM_ROWS, 128))


PAIRS = 4
CE = 800
NCH4 = N_EDGES // CE
GU = 10
M_HI = -65536


def _agg_body(vt_hbm, src_hbm, dst_hbm, w_hbm, agg_hbm,
              sl0, sl1, sl2, sl3, a0, a1, a2, a3, a4, a5, a6, a7,
              si0, di0, wv0, si1, di1, wv1, sem0, sem1, sems):
    cid = lax.axis_index("c")
    sid = lax.axis_index("s")
    wid = sid * NC + cid
    zero = jnp.zeros((L,), jnp.float32)
    slabs = (sl0, sl1, sl2, sl3)
    accs = (a0, a1, a2, a3, a4, a5, a6, a7)
    bufs = ((si0, di0, wv0, sem0), (si1, di1, wv1, sem1))

    def issue(i, b):
        si, di, wv, sem = bufs[b]
        off = i * CE
        pltpu.async_copy(src_hbm.at[pl.ds(off, CE)], si, sem)
        pltpu.async_copy(dst_hbm.at[pl.ds(off, CE)], di, sem)
        pltpu.async_copy(w_hbm.at[pl.ds(off, CE)], wv, sem)

    def wait(b):
        si, di, wv, sem = bufs[b]
        pltpu.make_async_copy(src_hbm.at[pl.ds(0, CE)], si, sem).wait()
        pltpu.make_async_copy(dst_hbm.at[pl.ds(0, CE)], di, sem).wait()
        pltpu.make_async_copy(w_hbm.at[pl.ds(0, CE)], wv, sem).wait()

    def compute(b):
        si, di, wv, _ = bufs[b]

        def grp(i, c2):
            for u in range(GU):
                g = i * GU + u
                s16 = si[pl.ds(g * L, L)]
                d16 = di[pl.ds(g * L, L)]
                w16 = wv[pl.ds(g * L, L)]
                for j in range(PAIRS):
                    word = plsc.load_gather(slabs[j], [s16])
                    fe = plsc.bitcast(lax.shift_left(word, 16), jnp.float32)
                    fo = plsc.bitcast(jnp.bitwise_and(word, jnp.int32(M_HI)), jnp.float32)
                    plsc.addupdate_scatter(accs[2 * j], [d16], fe * w16)
                    plsc.addupdate_scatter(accs[2 * j + 1], [d16], fo * w16)
            return c2

        lax.fori_loop(0, CE // (L * GU), grp, 0, unroll=False)

    cps = [pltpu.async_copy(vt_hbm.at[wid, pl.ds(j * NP, NP)], slabs[j], sems)
           for j in range(PAIRS)]
    issue(0, 0)

    def zinit(i, c):
        for j in range(2 * PAIRS):
            accs[j][pl.ds(i * L, L)] = zero
        return c

    lax.fori_loop(0, NP // L, zinit, 0, unroll=False)
    for cp in cps:
        cp.wait()

    def outer(i2, c):
        ia = 2 * i2
        issue(ia + 1, 1)
        wait(0)
        compute(0)

        @pl.when(ia + 2 < NCH4)
        def _():
            issue(ia + 2, 0)

        wait(1)
        compute(1)
        return c

    lax.fori_loop(0, NCH4 // 2, outer, 0, unroll=False)
    for jj in range(2 * PAIRS):
        pltpu.async_copy(accs[jj], agg_hbm.at[8 * wid + jj], sems)
    for jj in range(2 * PAIRS):
        pltpu.make_async_copy(accs[jj], agg_hbm.at[8 * wid + jj], sems).wait()


def _aggregate(vt_slabs, src, dst, w):
    return pl.kernel(
        _agg_body,
        out_type=jax.ShapeDtypeStruct((D, NP), jnp.float32),
        mesh=_sc_mesh(),
        compiler_params=_SC_PARAMS,
        scratch_types=[pltpu.VMEM((NP,), jnp.int32)] * PAIRS
                      + [pltpu.VMEM((NP,), jnp.float32)] * (2 * PAIRS)
                      + [
            pltpu.VMEM((CE,), jnp.int32),
            pltpu.VMEM((CE,), jnp.int32),
            pltpu.VMEM((CE,), jnp.float32),
            pltpu.VMEM((CE,), jnp.int32),
            pltpu.VMEM((CE,), jnp.int32),
            pltpu.VMEM((CE,), jnp.float32),
            pltpu.SemaphoreType.DMA,
            pltpu.SemaphoreType.DMA,
            pltpu.SemaphoreType.DMA,
        ],
    )(vt_slabs, src, dst, w)


def _final_body(agg_ref, emb_ref, out_ref):
    out_ref[...] = emb_ref[...] + lax.transpose(agg_ref[...], (1, 0))


def _finalize(agg_t, emb):
    R = 512
    return pl.pallas_call(
        _final_body,
        grid=(NP // R,),
        in_specs=[
            pl.BlockSpec((D, R), lambda i: (0, i)),
            pl.BlockSpec((R, D), lambda i: (i, 0)),
        ],
        out_specs=pl.BlockSpec((R, D), lambda i: (i, 0)),
        out_shape=jax.ShapeDtypeStruct((N_NODES, D), jnp.float32),
    )(agg_t, emb)


def kernel(embeddings, edge_index, Wq, bq, Wk, bk, Wv, bv):
    ws = (Wq[0::2], Wq[1::2], Wk[0::2], Wk[1::2], Wv[0::2], Wv[1::2])
    bs = tuple(b[i::2].reshape(D // 2, 1) for b in (bq, bk, bv) for i in (0, 1))

    edge3 = edge_index.astype(jnp.int32).reshape(2, EROWS, 128)
    qtp, ktp, vtp, src_p, dst_p = _project(embeddings, edge3, ws, bs)
    src_p = src_p.reshape(NE_PAD)
    dst_p = dst_p.reshape(NE_PAD)
    sparts = _scores(qtp.reshape(NW, PAIRS * NP), ktp.reshape(NW, PAIRS * NP),
                     dst_p, src_p)
    w = _softmax(sparts).reshape(NE_PAD)

    agg = _aggregate(vtp.reshape(NW, PAIRS * NP), src_p, dst_p, w)
    return _finalize(agg, embeddings)


# speedup vs baseline: 1.0116x; 1.0116x over previous
import jax
import jax.numpy as jnp
from jax import lax
from jax.experimental import pallas as pl
from jax.experimental.pallas import tpu as pltpu
from jax.experimental.pallas import tpu_sc as plsc

N_NODES = 10000
N_EDGES = 160000
D = 256
NC, NS, L = 2, 16, 16
NW = NC * NS
NP = 10240
NE_PAD = 163840

_SC_PARAMS = pltpu.CompilerParams(needs_layout_passes=False)


def _sc_mesh():
    return plsc.VectorSubcoreMesh(core_axis_name="c", subcore_axis_name="s",
                                  num_cores=NC, num_subcores=NS)


def _bf16_bits(x):
    return lax.bitcast_convert_type(x.astype(jnp.bfloat16).astype(jnp.float32),
                                    jnp.int32)


def _pack_pair(even, odd):
    lo = lax.shift_right_logical(_bf16_bits(even), 16)
    hi = jnp.bitwise_and(_bf16_bits(odd), jnp.int32(-65536))
    return jnp.bitwise_or(lo, hi)


EROWS = N_EDGES // 128
EROWS_P = NE_PAD // 128
ERB = EROWS_P // 20


def _proj_body(x_ref, e_ref, wqe_ref, wqo_ref, wke_ref, wko_ref, wve_ref, wvo_ref,
               bqe_ref, bqo_ref, bke_ref, bko_ref, bve_ref, bvo_ref,
               qt_ref, kt_ref, vt_ref, s_ref, d_ref):
    x = x_ref[...]
    inv = 1.0 / (D ** 0.5)
    dn = (((1,), (1,)), ((), ()))

    def mm(w_ref, b_ref):
        return lax.dot_general(w_ref[...], x, dn,
                               preferred_element_type=jnp.float32) + b_ref[...]

    qt_ref[...] = _pack_pair(mm(wqe_ref, bqe_ref) * inv, mm(wqo_ref, bqo_ref) * inv)
    kt_ref[...] = _pack_pair(mm(wke_ref, bke_ref), mm(wko_ref, bko_ref))
    vt_ref[...] = _pack_pair(mm(wve_ref, bve_ref), mm(wvo_ref, bvo_ref))
    i = pl.program_id(0)
    e = e_ref[...]
    valid = i * ERB + lax.broadcasted_iota(jnp.int32, (ERB, 128), 0) < EROWS
    s_ref[...] = jnp.where(valid, e[0], 0)
    d_ref[...] = jnp.where(valid, e[1], 0)


def _project(emb, edge3, ws, bs):
    R = 512
    half = pl.BlockSpec((D // 2, D), lambda i: (0, 0))
    bcol = pl.BlockSpec((D // 2, 1), lambda i: (0, 0))
    outs = pl.BlockSpec((D // 2, R), lambda i: (0, i))
    epad = pl.BlockSpec((ERB, 128), lambda i: (i, 0))
    return pl.pallas_call(
        _proj_body,
        grid=(NP // R,),
        in_specs=[pl.BlockSpec((R, D), lambda i: (i, 0)),
                  pl.BlockSpec((2, ERB, 128), lambda i: (0, i, 0))]
                 + [half] * 6 + [bcol] * 6,
        out_specs=[outs] * 3 + [epad] * 2,
        out_shape=[jax.ShapeDtypeStruct((D // 2, NP), jnp.int32)] * 3
                  + [jax.ShapeDtypeStruct((EROWS_P, 128), jnp.int32)] * 2,
    )(emb, edge3, *ws, *bs)


C2 = 2048
NCH2 = NE_PAD // C2
GU2 = 8


def _scores_body(qt_hbm, kt_hbm, dst_hbm, src_hbm, sp_hbm,
                 qslab, kslab, si0, di0, si1, di1, pb0, pb1,
                 semq, sem0, sem1, semp):
    cid = lax.axis_index("c")
    sid = lax.axis_index("s")
    wid = sid * NC + cid
    ibufs = ((si0, di0, sem0), (si1, di1, sem1))
    pbufs = (pb0, pb1)

    def issue(i, b):
        si, di, sem = ibufs[b]
        off = i * C2
        pltpu.async_copy(src_hbm.at[pl.ds(off, C2)], si, sem)
        pltpu.async_copy(dst_hbm.at[pl.ds(off, C2)], di, sem)

    def wait(b):
        si, di, sem = ibufs[b]
        pltpu.make_async_copy(src_hbm.at[pl.ds(0, C2)], si, sem).wait()
        pltpu.make_async_copy(dst_hbm.at[pl.ds(0, C2)], di, sem).wait()

    def compute(i, b):
        si, di, _ = ibufs[b]
        pbuf = pbufs[b]

        def grp(t, c2):
            for u in range(GU2):
                g = t * GU2 + u
                s16 = si[pl.ds(g * L, L)]
                d16 = di[pl.ds(g * L, L)]
                acc = jnp.zeros((L,), jnp.float32)
                for j in range(PAIRS):
                    qw = plsc.load_gather(qslab, [d16 + (j * NP)])
                    kw = plsc.load_gather(kslab, [s16 + (j * NP)])
                    qe = plsc.bitcast(lax.shift_left(qw, 16), jnp.float32)
                    ke = plsc.bitcast(lax.shift_left(kw, 16), jnp.float32)
                    qo = plsc.bitcast(jnp.bitwise_and(qw, jnp.int32(M_HI)), jnp.float32)
                    ko = plsc.bitcast(jnp.bitwise_and(kw, jnp.int32(M_HI)), jnp.float32)
                    acc = acc + qe * ke + qo * ko
                pbuf[pl.ds(g * L, L)] = acc
            return c2

        lax.fori_loop(0, C2 // (L * GU2), grp, 0, unroll=False)
        pltpu.async_copy(pbuf, sp_hbm.at[wid, pl.ds(i * C2, C2)], semp)

    def wait_pbuf(b):
        pltpu.make_async_copy(pbufs[b], sp_hbm.at[wid, pl.ds(0, C2)], semp).wait()

    cpq = pltpu.async_copy(qt_hbm.at[wid], qslab, semq)
    cpk = pltpu.async_copy(kt_hbm.at[wid], kslab, semq)
    issue(0, 0)
    cpq.wait()
    cpk.wait()

    def outer(i2, c):
        a = 2 * i2
        issue(a + 1, 1)
        wait(0)

        @pl.when(i2 > 0)
        def _():
            wait_pbuf(0)

        compute(a, 0)

        @pl.when(a + 2 < NCH2)
        def _():
            issue(a + 2, 0)

        wait(1)

        @pl.when(i2 > 0)
        def _():
            wait_pbuf(1)

        compute(a + 1, 1)
        return c

    lax.fori_loop(0, NCH2 // 2, outer, 0, unroll=False)
    wait_pbuf(0)
    wait_pbuf(1)


def _scores(qt_slabs, kt_slabs, dst, src):
    return pl.kernel(
        _scores_body,
        out_type=jax.ShapeDtypeStruct((NW, NE_PAD), jnp.float32),
        mesh=_sc_mesh(),
        compiler_params=_SC_PARAMS,
        scratch_types=[
            pltpu.VMEM((PAIRS * NP,), jnp.int32),
            pltpu.VMEM((PAIRS * NP,), jnp.int32),
            pltpu.VMEM((C2,), jnp.int32),
            pltpu.VMEM((C2,), jnp.int32),
            pltpu.VMEM((C2,), jnp.int32),
            pltpu.VMEM((C2,), jnp.int32),
            pltpu.VMEM((C2,), jnp.float32),
            pltpu.VMEM((C2,), jnp.float32),
            pltpu.SemaphoreType.DMA,
            pltpu.SemaphoreType.DMA,
            pltpu.SemaphoreType.DMA,
            pltpu.SemaphoreType.DMA,
        ],
    )(qt_slabs, kt_slabs, dst, src)


SM_ROWS = NE_PAD // 128


def _softmax_body(sp_ref, w_ref):
    s = jnp.sum(sp_ref[...], axis=0)
    rows = lax.broadcasted_iota(jnp.int32, (SM_ROWS, 128), 0)
    cols = lax.broadcasted_iota(jnp.int32, (SM_ROWS, 128), 1)
    valid = rows * 128 + cols < N_EDGES
    s = jnp.where(valid, s, -jnp.inf)
    m = jnp.max(s)
    e = jnp.where(valid, jnp.exp(s - m), 0.0)
    w_ref[...] = e / jnp.sum(e)


def _softmax(sparts):
    return pl.pallas_call(
        _softmax_body,
        out_shape=jax.ShapeDtypeStruct((SM_ROWS, 128), jnp.float32),
    )(sparts.reshape(NW, SM_ROWS, 128))


PAIRS = 4
CE = 800
NCH4 = N_EDGES // CE
GU = 10
M_HI = -65536


def _agg_body(vt_hbm, src_hbm, dst_hbm, w_hbm, agg_hbm,
              slab, acc, si0, di0, wv0, si1, di1, wv1, sem0, sem1, sems):
    cid = lax.axis_index("c")
    sid = lax.axis_index("s")
    wid = sid * NC + cid
    zero = jnp.zeros((L,), jnp.float32)
    bufs = ((si0, di0, wv0, sem0), (si1, di1, wv1, sem1))

    def issue(i, b):
        si, di, wv, sem = bufs[b]
        off = i * CE
        pltpu.async_copy(src_hbm.at[pl.ds(off, CE)], si, sem)
        pltpu.async_copy(dst_hbm.at[pl.ds(off, CE)], di, sem)
        pltpu.async_copy(w_hbm.at[pl.ds(off, CE)], wv, sem)

    def wait(b):
        si, di, wv, sem = bufs[b]
        pltpu.make_async_copy(src_hbm.at[pl.ds(0, CE)], si, sem).wait()
        pltpu.make_async_copy(dst_hbm.at[pl.ds(0, CE)], di, sem).wait()
        pltpu.make_async_copy(w_hbm.at[pl.ds(0, CE)], wv, sem).wait()

    def compute(b):
        si, di, wv, _ = bufs[b]

        def grp(i, c2):
            for u in range(GU):
                g = i * GU + u
                s16 = si[pl.ds(g * L, L)]
                d16 = di[pl.ds(g * L, L)]
                w16 = wv[pl.ds(g * L, L)]
                for j in range(PAIRS):
                    word = plsc.load_gather(slab, [s16 + (j * NP)])
                    fe = plsc.bitcast(lax.shift_left(word, 16), jnp.float32)
                    fo = plsc.bitcast(jnp.bitwise_and(word, jnp.int32(M_HI)), jnp.float32)
                    plsc.addupdate_scatter(acc, [d16 + (2 * j * NP)], fe * w16)
                    plsc.addupdate_scatter(acc, [d16 + ((2 * j + 1) * NP)], fo * w16)
            return c2

        lax.fori_loop(0, CE // (L * GU), grp, 0, unroll=False)

    cp_slab = pltpu.async_copy(vt_hbm.at[wid], slab, sems)
    issue(0, 0)

    def zinit(i, c):
        for j in range(2 * PAIRS):
            acc[pl.ds(j * NP + i * L, L)] = zero
        return c

    lax.fori_loop(0, NP // L, zinit, 0, unroll=False)
    cp_slab.wait()

    def outer(i2, c):
        ia = 2 * i2
        issue(ia + 1, 1)
        wait(0)
        compute(0)

        @pl.when(ia + 2 < NCH4)
        def _():
            issue(ia + 2, 0)

        wait(1)
        compute(1)
        return c

    lax.fori_loop(0, NCH4 // 2, outer, 0, unroll=False)
    for jj in range(2 * PAIRS):
        pltpu.async_copy(acc.at[pl.ds(jj * NP, NP)], agg_hbm.at[8 * wid + jj], sems)
    for jj in range(2 * PAIRS):
        pltpu.make_async_copy(acc.at[pl.ds(jj * NP, NP)], agg_hbm.at[8 * wid + jj],
                              sems).wait()


def _aggregate(vt_slabs, src, dst, w):
    return pl.kernel(
        _agg_body,
        out_type=jax.ShapeDtypeStruct((D, NP), jnp.float32),
        mesh=_sc_mesh(),
        compiler_params=_SC_PARAMS,
        scratch_types=[
            pltpu.VMEM((PAIRS * NP,), jnp.int32),
            pltpu.VMEM((2 * PAIRS * NP,), jnp.float32),
            pltpu.VMEM((CE,), jnp.int32),
            pltpu.VMEM((CE,), jnp.int32),
            pltpu.VMEM((CE,), jnp.float32),
            pltpu.VMEM((CE,), jnp.int32),
            pltpu.VMEM((CE,), jnp.int32),
            pltpu.VMEM((CE,), jnp.float32),
            pltpu.SemaphoreType.DMA,
            pltpu.SemaphoreType.DMA,
            pltpu.SemaphoreType.DMA,
        ],
    )(vt_slabs, src, dst, w)


def _final_body(agg_ref, emb_ref, out_ref):
    out_ref[...] = emb_ref[...] + lax.transpose(agg_ref[...], (1, 0))


def _finalize(agg_t, emb):
    R = 512
    return pl.pallas_call(
        _final_body,
        grid=(NP // R,),
        in_specs=[
            pl.BlockSpec((D, R), lambda i: (0, i)),
            pl.BlockSpec((R, D), lambda i: (i, 0)),
        ],
        out_specs=pl.BlockSpec((R, D), lambda i: (i, 0)),
        out_shape=jax.ShapeDtypeStruct((N_NODES, D), jnp.float32),
    )(agg_t, emb)


def kernel(embeddings, edge_index, Wq, bq, Wk, bk, Wv, bv):
    ws = (Wq[0::2], Wq[1::2], Wk[0::2], Wk[1::2], Wv[0::2], Wv[1::2])
    bs = tuple(b[i::2].reshape(D // 2, 1) for b in (bq, bk, bv) for i in (0, 1))

    edge3 = edge_index.astype(jnp.int32).reshape(2, EROWS, 128)
    qtp, ktp, vtp, src_p, dst_p = _project(embeddings, edge3, ws, bs)
    src_p = src_p.reshape(NE_PAD)
    dst_p = dst_p.reshape(NE_PAD)
    sparts = _scores(qtp.reshape(NW, PAIRS * NP), ktp.reshape(NW, PAIRS * NP),
                     dst_p, src_p)
    w = _softmax(sparts).reshape(NE_PAD)

    agg = _aggregate(vtp.reshape(NW, PAIRS * NP), src_p, dst_p, w)
    return _finalize(agg, embeddings)
